# baseline (speedup 1.0000x reference)
import functools

import jax
import jax.numpy as jnp
from jax import lax
from jax.experimental import pallas as pl
from jax.experimental.pallas import tpu as pltpu
from jax.experimental.pallas import tpu_sc as plsc

BATCH = 16384
EMBED = 64
NUM_CORES = 2
NUM_SUBCORES = 16
NUM_WORKERS = NUM_CORES * NUM_SUBCORES
B_PER_W = BATCH // NUM_WORKERS
CHUNK = 128
NCHUNK = B_PER_W // CHUNK
GROUPS = CHUNK // 16

_mesh = plsc.VectorSubcoreMesh(core_axis_name="c", subcore_axis_name="s")


@functools.partial(
    pl.kernel,
    out_type=jax.ShapeDtypeStruct((BATCH,), jnp.float32),
    mesh=_mesh,
    compiler_params=pltpu.CompilerParams(
        needs_layout_passes=False, use_tc_tiling_on_sc=False),
    scratch_types=[
        pltpu.VMEM((NCHUNK, CHUNK), jnp.int32),
        [pltpu.VMEM((CHUNK, EMBED), jnp.float32) for _ in range(NCHUNK)],
        pltpu.VMEM((B_PER_W,), jnp.float32),
        pltpu.VMEM((1, EMBED), jnp.float32),
        pltpu.VMEM((16,), jnp.float32),
        pltpu.SemaphoreType.DMA,
    ],
)
def _sc_kernel(ids_hbm, table_hbm, w_hbm, b_hbm, out_hbm,
               idx_v, rows_v, out_v, w_v, b_v, sem):
    wid = lax.axis_index("s") * NUM_CORES + lax.axis_index("c")
    base = pl.multiple_of(wid * B_PER_W, B_PER_W)

    pltpu.sync_copy(w_hbm, w_v)
    pltpu.sync_copy(b_hbm, b_v.at[pl.ds(0, 1)])
    for c in range(NCHUNK):
        pltpu.sync_copy(ids_hbm.at[pl.ds(base + c * CHUNK, CHUNK)],
                        idx_v.at[c])

    copies = []
    for c in range(NCHUNK):
        copies.append(
            pltpu.async_copy(table_hbm.at[idx_v.at[c]], rows_v[c], sem))
    for cp in copies:
        cp.wait()

    b_s = b_v[pl.ds(0, 16)][0]
    w_vecs = [w_v[0, pl.ds(k * 16, 16)] for k in range(EMBED // 16)]
    w_s = [w_vecs[d // 16][d % 16] for d in range(EMBED)]
    lane = lax.iota(jnp.int32, 16)

    for c in range(NCHUNK):
        rows_c = rows_v[c]

        def body(g, _, rows_c=rows_c):
            row_ids = g * 16 + lane
            acc = jnp.full((16,), b_s, jnp.float32)
            for d in range(EMBED):
                col = jnp.full((16,), d, jnp.int32)
                v = plsc.load_gather(rows_c, [row_ids, col])
                acc = acc + v * w_s[d]
            off = pl.multiple_of(c * CHUNK + g * 16, 16)
            out_v[pl.ds(off, 16)] = acc
            return 0

        lax.fori_loop(0, GROUPS, body, 0)

    pltpu.sync_copy(out_v, out_hbm.at[pl.ds(base, B_PER_W)])


def kernel(dataset_ids, table, W, b):
    return _sc_kernel(dataset_ids.astype(jnp.int32), table, W, b)
